# initial kernel scaffold (unmeasured)
import jax
import jax.numpy as jnp
from jax import lax
from jax.experimental import pallas as pl
from jax.experimental.pallas import tpu as pltpu

NZ = 4
T = 512
TP = T // NZ
D = 512
F = 1024
EP = 2
E = NZ * EP


def kernel(x, router, W1, W2):
    assert x.shape == (TP, D), x.shape
    assert router.shape == (T, EP), router.shape
    assert W1.shape == (EP, D, F), W1.shape
    assert W2.shape == (EP, F, D), W2.shape

    def body(x_ref, r_ref, w1_ref, w2_ref, out_ref,
             xall, rall, part, rs_send, rs_recv,
             ag_send_sems, ag_recv_sems,
             rt_send_sems, rt_recv_sems,
             rs_send_sems, rs_recv_sems):
        my_x = lax.axis_index("x")
        my_y = lax.axis_index("y")
        my_z = lax.axis_index("z")
        left = (my_x, my_y, (my_z - 1) % NZ)
        right = (my_x, my_y, (my_z + 1) % NZ)

        barrier = pltpu.get_barrier_semaphore()
        for nbr in (left, right):
            pl.semaphore_signal(barrier, inc=1, device_id=nbr,
                                device_id_type=pl.DeviceIdType.MESH)
        pl.semaphore_wait(barrier, 2)

        xall[pl.ds(my_z, 1)] = x_ref[...][None]
        rall[pl.ds(my_z, 1)] = jnp.transpose(r_ref[...])[None]

        for h in range(NZ - 1):
            slot = (my_z - h) % NZ
            x_rdma = pltpu.make_async_remote_copy(
                src_ref=xall.at[slot], dst_ref=xall.at[slot],
                send_sem=ag_send_sems.at[h], recv_sem=ag_recv_sems.at[h],
                device_id=right, device_id_type=pl.DeviceIdType.MESH)
            r_rdma = pltpu.make_async_remote_copy(
                src_ref=rall.at[slot], dst_ref=rall.at[slot],
                send_sem=rt_send_sems.at[h], recv_sem=rt_recv_sems.at[h],
                device_id=right, device_id_type=pl.DeviceIdType.MESH)
            x_rdma.start()
            r_rdma.start()
            x_rdma.wait()
            r_rdma.wait()

        xa = jnp.reshape(xall[...], (T, D))
        r8 = jnp.reshape(rall[...], (E, T))
        gates = lax.dot_general(
            xa, r8, (((1,), (1,)), ((), ())),
            preferred_element_type=jnp.float32)

        v1 = jnp.max(gates, axis=1, keepdims=True)
        i1 = jnp.argmax(gates, axis=1, keepdims=True)
        eids = lax.broadcasted_iota(jnp.int32, (T, E), 1)
        masked = jnp.where(eids == i1, -1e30, gates)
        v2 = jnp.max(masked, axis=1, keepdims=True)
        i2 = jnp.argmax(masked, axis=1, keepdims=True)
        w_top1 = 1.0 / (1.0 + jnp.exp(v2 - v1))
        w_top2 = 1.0 - w_top1

        acc = jnp.zeros((T, D), jnp.float32)
        for j in range(EP):
            e = my_z * EP + j
            we = (jnp.where(i1 == e, w_top1, 0.0)
                  + jnp.where(i2 == e, w_top2, 0.0))
            h1 = jnp.maximum(
                jnp.dot(xa, w1_ref[j], preferred_element_type=jnp.float32),
                0.0)
            acc = acc + we * jnp.dot(
                h1, w2_ref[j], preferred_element_type=jnp.float32)
        part[...] = acc

        for s in range(NZ - 1):
            c = (my_z - s - 1) % NZ
            chunk = part[pl.ds(c * TP, TP), :]
            if s == 0:
                val = chunk
            else:
                val = rs_recv[s - 1] + chunk
            rs_send[s] = val
            rdma = pltpu.make_async_remote_copy(
                src_ref=rs_send.at[s], dst_ref=rs_recv.at[s],
                send_sem=rs_send_sems.at[s], recv_sem=rs_recv_sems.at[s],
                device_id=right, device_id_type=pl.DeviceIdType.MESH)
            rdma.start()
            rdma.wait()
        out_ref[...] = rs_recv[NZ - 2] + part[pl.ds(my_z * TP, TP), :]

    return pl.pallas_call(
        body,
        out_shape=jax.ShapeDtypeStruct((TP, D), jnp.float32),
        in_specs=[pl.BlockSpec(memory_space=pltpu.VMEM)] * 4,
        out_specs=pl.BlockSpec(memory_space=pltpu.VMEM),
        scratch_shapes=[
            pltpu.VMEM((NZ, TP, D), jnp.float32),
            pltpu.VMEM((NZ, EP, T), jnp.float32),
            pltpu.VMEM((T, D), jnp.float32),
            pltpu.VMEM((NZ - 1, TP, D), jnp.float32),
            pltpu.VMEM((NZ - 1, TP, D), jnp.float32),
            pltpu.SemaphoreType.DMA((NZ - 1,)),
            pltpu.SemaphoreType.DMA((NZ - 1,)),
            pltpu.SemaphoreType.DMA((NZ - 1,)),
            pltpu.SemaphoreType.DMA((NZ - 1,)),
            pltpu.SemaphoreType.DMA((NZ - 1,)),
            pltpu.SemaphoreType.DMA((NZ - 1,)),
        ],
        compiler_params=pltpu.CompilerParams(collective_id=0),
    )(x, router, W1, W2)


# baseline (device time: 42997 ns/iter reference)
import jax
import jax.numpy as jnp
from jax import lax
from jax.experimental import pallas as pl
from jax.experimental.pallas import tpu as pltpu

NZ = 4
T = 512
TP = T // NZ
D = 512
F = 1024
EP = 2
E = NZ * EP


def kernel(x, router, W1, W2):
    assert x.shape == (TP, D), x.shape
    assert router.shape == (T, EP), router.shape
    assert W1.shape == (EP, D, F), W1.shape
    assert W2.shape == (EP, F, D), W2.shape

    def body(x_ref, r_ref, w1_ref, w2_ref, out_ref,
             xall, rall, part, rs_send, rs_recv,
             ag_send_sems, ag_recv_sems,
             rt_send_sems, rt_recv_sems,
             rs_send_sems, rs_recv_sems):
        my_x = lax.axis_index("x")
        my_y = lax.axis_index("y")
        my_z = lax.axis_index("z")
        left = (my_x, my_y, (my_z - 1) % NZ)
        right = (my_x, my_y, (my_z + 1) % NZ)

        barrier = pltpu.get_barrier_semaphore()
        for nbr in (left, right):
            pl.semaphore_signal(barrier, inc=1, device_id=nbr,
                                device_id_type=pl.DeviceIdType.MESH)
        pl.semaphore_wait(barrier, 2)

        xall[pl.ds(my_z, 1)] = x_ref[...][None]
        rall[pl.ds(my_z, 1)] = jnp.transpose(r_ref[...])[None]

        for h in range(NZ - 1):
            slot = (my_z - h) % NZ
            x_rdma = pltpu.make_async_remote_copy(
                src_ref=xall.at[slot], dst_ref=xall.at[slot],
                send_sem=ag_send_sems.at[h], recv_sem=ag_recv_sems.at[h],
                device_id=right, device_id_type=pl.DeviceIdType.MESH)
            r_rdma = pltpu.make_async_remote_copy(
                src_ref=rall.at[slot], dst_ref=rall.at[slot],
                send_sem=rt_send_sems.at[h], recv_sem=rt_recv_sems.at[h],
                device_id=right, device_id_type=pl.DeviceIdType.MESH)
            x_rdma.start()
            r_rdma.start()
            x_rdma.wait()
            r_rdma.wait()

        xa = jnp.reshape(xall[...], (T, D))
        r8 = jnp.reshape(rall[...], (E, T))
        gates = lax.dot_general(
            xa, r8, (((1,), (1,)), ((), ())),
            precision=lax.Precision.HIGHEST,
            preferred_element_type=jnp.float32)

        v1 = jnp.max(gates, axis=1, keepdims=True)
        i1 = jnp.argmax(gates, axis=1, keepdims=True)
        eids = lax.broadcasted_iota(jnp.int32, (T, E), 1)
        masked = jnp.where(eids == i1, -1e30, gates)
        v2 = jnp.max(masked, axis=1, keepdims=True)
        i2 = jnp.argmax(masked, axis=1, keepdims=True)
        w_top1 = 1.0 / (1.0 + jnp.exp(v2 - v1))
        w_top2 = 1.0 - w_top1

        acc = jnp.zeros((T, D), jnp.float32)
        for j in range(EP):
            e = my_z * EP + j
            we = (jnp.where(i1 == e, w_top1, 0.0)
                  + jnp.where(i2 == e, w_top2, 0.0))
            h1 = jnp.maximum(
                jnp.dot(xa, w1_ref[j], preferred_element_type=jnp.float32),
                0.0)
            acc = acc + we * jnp.dot(
                h1, w2_ref[j], preferred_element_type=jnp.float32)
        part[...] = acc

        for s in range(NZ - 1):
            c = (my_z - s - 1) % NZ
            chunk = part[pl.ds(c * TP, TP), :]
            if s == 0:
                val = chunk
            else:
                val = rs_recv[s - 1] + chunk
            rs_send[s] = val
            rdma = pltpu.make_async_remote_copy(
                src_ref=rs_send.at[s], dst_ref=rs_recv.at[s],
                send_sem=rs_send_sems.at[s], recv_sem=rs_recv_sems.at[s],
                device_id=right, device_id_type=pl.DeviceIdType.MESH)
            rdma.start()
            rdma.wait()
        out_ref[...] = rs_recv[NZ - 2] + part[pl.ds(my_z * TP, TP), :]

    return pl.pallas_call(
        body,
        out_shape=jax.ShapeDtypeStruct((TP, D), jnp.float32),
        in_specs=[pl.BlockSpec(memory_space=pltpu.VMEM)] * 4,
        out_specs=pl.BlockSpec(memory_space=pltpu.VMEM),
        scratch_shapes=[
            pltpu.VMEM((NZ, TP, D), jnp.float32),
            pltpu.VMEM((NZ, EP, T), jnp.float32),
            pltpu.VMEM((T, D), jnp.float32),
            pltpu.VMEM((NZ - 1, TP, D), jnp.float32),
            pltpu.VMEM((NZ - 1, TP, D), jnp.float32),
            pltpu.SemaphoreType.DMA((NZ - 1,)),
            pltpu.SemaphoreType.DMA((NZ - 1,)),
            pltpu.SemaphoreType.DMA((NZ - 1,)),
            pltpu.SemaphoreType.DMA((NZ - 1,)),
            pltpu.SemaphoreType.DMA((NZ - 1,)),
            pltpu.SemaphoreType.DMA((NZ - 1,)),
        ],
        compiler_params=pltpu.CompilerParams(collective_id=0),
    )(x, router, W1, W2)


# device time: 28090 ns/iter; 1.5307x vs baseline; 1.5307x over previous
import jax
import jax.numpy as jnp
from jax import lax
from jax.experimental import pallas as pl
from jax.experimental.pallas import tpu as pltpu

NZ = 4
T = 512
TP = T // NZ
D = 512
F = 1024
EP = 2
E = NZ * EP


def kernel(x, router, W1, W2):
    assert x.shape == (TP, D), x.shape
    assert router.shape == (T, EP), router.shape
    assert W1.shape == (EP, D, F), W1.shape
    assert W2.shape == (EP, F, D), W2.shape

    def body(x_ref, r_ref, w1_ref, w2_ref, out_ref,
             xall, rall, wall, psend, precv,
             x_send_sems, x_recv_sems,
             rt_send_sems, rt_recv_sems,
             wt_send_sems, wt_recv_sems,
             p_send_sems, p_recv_sems):
        my_x = lax.axis_index("x")
        my_y = lax.axis_index("y")
        my_z = lax.axis_index("z")

        def peer(k):
            return (my_x, my_y, (my_z + k) % NZ)

        precv[pl.ds(my_z, 1)] = jnp.zeros((1, TP, D), jnp.bfloat16)

        barrier = pltpu.get_barrier_semaphore()
        for k in range(1, NZ):
            pl.semaphore_signal(barrier, inc=1, device_id=peer(k),
                                device_id_type=pl.DeviceIdType.MESH)
        pl.semaphore_wait(barrier, NZ - 1)

        xbf = x_ref[...].astype(jnp.bfloat16)
        xall[pl.ds(my_z, 1)] = xbf[None]
        rall[pl.ds(my_z, 1)] = jnp.transpose(r_ref[...])[None]

        x_sends = []
        rt_sends = []
        for k in range(1, NZ):
            tgt = (my_z + k) % NZ
            s = pltpu.make_async_remote_copy(
                src_ref=xall.at[my_z], dst_ref=xall.at[my_z],
                send_sem=x_send_sems.at[tgt], recv_sem=x_recv_sems.at[my_z],
                device_id=peer(k), device_id_type=pl.DeviceIdType.MESH)
            s.start()
            x_sends.append(s)
            s = pltpu.make_async_remote_copy(
                src_ref=rall.at[my_z], dst_ref=rall.at[my_z],
                send_sem=rt_send_sems.at[tgt], recv_sem=rt_recv_sems.at[my_z],
                device_id=peer(k), device_id_type=pl.DeviceIdType.MESH)
            s.start()
            rt_sends.append(s)

        w1b0 = w1_ref[0].astype(jnp.bfloat16)
        w1b1 = w1_ref[1].astype(jnp.bfloat16)
        w2b0 = w2_ref[0].astype(jnp.bfloat16)
        w2b1 = w2_ref[1].astype(jnp.bfloat16)

        def ffn(xc):
            h0 = jnp.maximum(
                jnp.dot(xc, w1b0, preferred_element_type=jnp.float32), 0.0)
            y0 = jnp.dot(h0.astype(jnp.bfloat16), w2b0,
                         preferred_element_type=jnp.float32)
            h1 = jnp.maximum(
                jnp.dot(xc, w1b1, preferred_element_type=jnp.float32), 0.0)
            y1 = jnp.dot(h1.astype(jnp.bfloat16), w2b1,
                         preferred_element_type=jnp.float32)
            return y0, y1

        y_own0, y_own1 = ffn(xbf)

        for k in range(1, NZ):
            c = (my_z + k) % NZ
            recv = pltpu.make_async_remote_copy(
                src_ref=rall.at[c], dst_ref=rall.at[c],
                send_sem=rt_send_sems.at[c], recv_sem=rt_recv_sems.at[c],
                device_id=peer(k), device_id_type=pl.DeviceIdType.MESH)
            recv.wait_recv()
        r8 = jnp.reshape(rall[...], (E, T))
        gates = lax.dot_general(
            x_ref[...], r8, (((1,), (1,)), ((), ())),
            precision=lax.Precision.HIGHEST,
            preferred_element_type=jnp.float32)
        v1 = jnp.max(gates, axis=1, keepdims=True)
        i1 = jnp.argmax(gates, axis=1, keepdims=True)
        eids = lax.broadcasted_iota(jnp.int32, (TP, E), 1)
        masked = jnp.where(eids == i1, -1e30, gates)
        v2 = jnp.max(masked, axis=1, keepdims=True)
        i2 = jnp.argmax(masked, axis=1, keepdims=True)
        w_top1 = 1.0 / (1.0 + jnp.exp(v2 - v1))
        w_top2 = 1.0 - w_top1
        we = (jnp.where(eids == i1, w_top1, 0.0)
              + jnp.where(eids == i2, w_top2, 0.0))

        wall[pl.ds(my_z, 1)] = jnp.transpose(we)[None]
        wt_sends = []
        for k in range(1, NZ):
            tgt = (my_z + k) % NZ
            s = pltpu.make_async_remote_copy(
                src_ref=wall.at[my_z], dst_ref=wall.at[my_z],
                send_sem=wt_send_sems.at[tgt], recv_sem=wt_recv_sems.at[my_z],
                device_id=peer(k), device_id_type=pl.DeviceIdType.MESH)
            s.start()
            wt_sends.append(s)

        e0 = my_z * EP
        e1 = my_z * EP + 1

        def weigh(we_mat, y0, y1):
            c0 = jnp.sum(jnp.where(eids == e0, we_mat, 0.0), axis=1,
                         keepdims=True)
            c1 = jnp.sum(jnp.where(eids == e1, we_mat, 0.0), axis=1,
                         keepdims=True)
            return c0 * y0 + c1 * y1

        part_own = weigh(we, y_own0, y_own1)

        p_sends = []
        for k in range(1, NZ):
            c = (my_z + k) % NZ
            xrecv = pltpu.make_async_remote_copy(
                src_ref=xall.at[c], dst_ref=xall.at[c],
                send_sem=x_send_sems.at[c], recv_sem=x_recv_sems.at[c],
                device_id=peer(k), device_id_type=pl.DeviceIdType.MESH)
            xrecv.wait_recv()
            xc = jnp.reshape(xall[pl.ds(c, 1)], (TP, D))
            yc0, yc1 = ffn(xc)
            wrecv = pltpu.make_async_remote_copy(
                src_ref=wall.at[c], dst_ref=wall.at[c],
                send_sem=wt_send_sems.at[c], recv_sem=wt_recv_sems.at[c],
                device_id=peer(k), device_id_type=pl.DeviceIdType.MESH)
            wrecv.wait_recv()
            we_c = jnp.transpose(jnp.reshape(wall[pl.ds(c, 1)], (E, TP)))
            part_c = weigh(we_c, yc0, yc1)
            psend[pl.ds(c, 1)] = part_c.astype(jnp.bfloat16)[None]
            s = pltpu.make_async_remote_copy(
                src_ref=psend.at[c], dst_ref=precv.at[my_z],
                send_sem=p_send_sems.at[c], recv_sem=p_recv_sems.at[my_z],
                device_id=peer(k), device_id_type=pl.DeviceIdType.MESH)
            s.start()
            p_sends.append(s)

        for k in range(1, NZ):
            c = (my_z + k) % NZ
            precv_wait = pltpu.make_async_remote_copy(
                src_ref=precv.at[c], dst_ref=precv.at[c],
                send_sem=p_send_sems.at[c], recv_sem=p_recv_sems.at[c],
                device_id=peer(k), device_id_type=pl.DeviceIdType.MESH)
            precv_wait.wait_recv()
        out_ref[...] = part_own + jnp.sum(
            precv[...].astype(jnp.float32), axis=0)

        for s in x_sends + rt_sends + wt_sends + p_sends:
            s.wait_send()

    return pl.pallas_call(
        body,
        out_shape=jax.ShapeDtypeStruct((TP, D), jnp.float32),
        in_specs=[pl.BlockSpec(memory_space=pltpu.VMEM)] * 4,
        out_specs=pl.BlockSpec(memory_space=pltpu.VMEM),
        scratch_shapes=[
            pltpu.VMEM((NZ, TP, D), jnp.bfloat16),
            pltpu.VMEM((NZ, EP, T), jnp.float32),
            pltpu.VMEM((NZ, E, TP), jnp.float32),
            pltpu.VMEM((NZ, TP, D), jnp.bfloat16),
            pltpu.VMEM((NZ, TP, D), jnp.bfloat16),
            pltpu.SemaphoreType.DMA((NZ,)),
            pltpu.SemaphoreType.DMA((NZ,)),
            pltpu.SemaphoreType.DMA((NZ,)),
            pltpu.SemaphoreType.DMA((NZ,)),
            pltpu.SemaphoreType.DMA((NZ,)),
            pltpu.SemaphoreType.DMA((NZ,)),
            pltpu.SemaphoreType.DMA((NZ,)),
            pltpu.SemaphoreType.DMA((NZ,)),
        ],
        compiler_params=pltpu.CompilerParams(collective_id=0),
    )(x, router, W1, W2)
